# baseline (speedup 1.0000x reference)
import functools

import jax
import jax.numpy as jnp
from jax import lax
from jax.experimental import pallas as pl
from jax.experimental.pallas import tpu as pltpu

_TAPS = ((0, 0), (0, 1), (1, 0), (1, 1))
_WSP = 16


def _wprep_kernel(w_ref, o_ref, *, chunk):
    wb = w_ref[...].astype(jnp.bfloat16)
    r = lax.broadcasted_iota(jnp.int32, (chunk, chunk), 0)
    c = lax.broadcasted_iota(jnp.int32, (chunk, chunk), 1)
    eye = (r == c).astype(jnp.bfloat16)
    wt = lax.dot_general(wb, eye, (((0,), (0,)), ((), ())),
                         preferred_element_type=jnp.float32)
    v = wt.astype(jnp.bfloat16).reshape(-1, 16, chunk)
    for g in range(16):
        o_ref[g] = v[:, g, :]


def _weight_slabs(w2d, rows, inner_nc, chunk=128):
    chunk = min(chunk, inner_nc)
    return pl.pallas_call(
        functools.partial(_wprep_kernel, chunk=chunk),
        out_shape=jax.ShapeDtypeStruct((16, rows, inner_nc), jnp.bfloat16),
        grid=(inner_nc // chunk,),
        in_specs=[pl.BlockSpec((chunk, rows * 16), lambda k: (k, 0))],
        out_specs=pl.BlockSpec((16, rows, chunk), lambda k: (0, 0, k)),
        compiler_params=pltpu.CompilerParams(
            dimension_semantics=("parallel",)),
    )(w2d)


def _core_kernel(x_ref, ps_ref, wd_ref, wu_ref, p_ref,
                 z_ref, zsum_ref, zssq_ref, xs_ref, ypad_ref,
                 *, NB, Ho, Wo, C_in, inner_nc):
    HoWo = Ho * Wo
    Hs = Ho + 1

    xb = x_ref[...].astype(jnp.float32)
    xb = jnp.where(xb > 0, xb, 0.2 * xb).astype(jnp.bfloat16)
    dn = (((1,), (1,)), ((), ()))
    for n in range(NB):
        xt = lax.dot_general(ps_ref[...], xb[n], dn,
                             preferred_element_type=jnp.float32)
        xs_ref[n] = xt.astype(jnp.bfloat16).reshape(4, Hs, _WSP, C_in)

    y = None
    for dy, dx in _TAPS:
        for py, px in _TAPS:
            g = (2 * dy + py) * 4 + (2 * dx + px)
            win = xs_ref[:, py * 2 + px, dy:dy + Ho, dx:dx + Wo, :].reshape(
                NB * HoWo, C_in)
            acc = jnp.dot(win, wd_ref[g], preferred_element_type=jnp.float32)
            y = acc if y is None else y + acc
    y = jnp.maximum(y, 0.0).astype(jnp.bfloat16)

    ypad_ref[...] = jnp.zeros(ypad_ref.shape, ypad_ref.dtype)
    ypad_ref[:, 1:1 + Ho, 1:1 + Wo, :] = y.reshape(NB, Ho, Wo, inner_nc)

    S = [[ypad_ref[:, r:r + Ho, s:s + Wo, :].reshape(NB * HoWo, inner_nc)
          for s in range(3)] for r in range(3)]

    zphs = []
    for ph, (py, px) in enumerate(_TAPS):
        zph = None
        for dy, dx in _TAPS:
            g = (3 - py - 2 * dy) * 4 + (3 - px - 2 * dx)
            acc = lax.dot_general(wu_ref[g], S[py + dy][px + dx], dn,
                                  preferred_element_type=jnp.float32)
            zph = acc if zph is None else zph + acc
        zphs.append(zph.astype(jnp.bfloat16))

    for n in range(NB):
        zc = None
        for ph in range(4):
            zn = zphs[ph][:, n * HoWo:(n + 1) * HoWo]
            acc = jnp.dot(zn, p_ref[ph], preferred_element_type=jnp.float32)
            zc = acc if zc is None else zc + acc
        z_ref[n] = zc.astype(jnp.bfloat16)
        zsum_ref[n] = jnp.sum(zc, axis=1, keepdims=True)
        zssq_ref[n] = jnp.sum(zc * zc, axis=1, keepdims=True)


def _bn_concat_kernel(x_ref, z_ref, scale_ref, shift_ref, o_ref, *, C_in):
    o_ref[:, :C_in, :] = x_ref[...]
    o_ref[:, C_in:, :] = (z_ref[...].astype(jnp.float32)
                          * scale_ref[...] + shift_ref[...])


def kernel(x_nchw, w_down, w_up, gamma, beta):
    eps = 1e-5
    N, C_in, H, W = x_nchw.shape
    inner_nc = w_down.shape[0]
    outer_nc = w_up.shape[1]
    Ho, Wo = H // 2, W // 2
    Hs = Ho + 1
    HW = H * W
    HoWo = Ho * Wo
    NB = 8 if N % 8 == 0 else (4 if N % 4 == 0 else 1)

    x_flat = x_nchw.astype(jnp.float32).reshape(N, C_in, HW)

    pyx = jnp.arange(4)[:, None, None]
    hs_i = jnp.arange(Hs)[None, :, None]
    ws_i = jnp.arange(_WSP)[None, None, :]
    h_src = 2 * hs_i + pyx // 2 - 1
    w_src = 2 * ws_i + pyx % 2 - 1
    valid = ((h_src >= 0) & (h_src < H) & (w_src >= 0) & (w_src < W)
             & (ws_i < Hs))
    src = jnp.where(valid, h_src * W + w_src, -1).reshape(-1, 1)
    ps2d = (src == jnp.arange(HW)[None, :]).astype(jnp.bfloat16)

    wd_slabs = _weight_slabs(w_down.reshape(inner_nc, C_in * 16),
                             C_in, inner_nc)
    wu_slabs = _weight_slabs(w_up.reshape(inner_nc, outer_nc * 16),
                             outer_nc, inner_nc)

    by = jnp.arange(Ho)[:, None]
    bx = jnp.arange(Wo)[None, :]
    lane = jnp.arange(HW)[None, :]
    pmats = []
    for py in range(2):
        for px in range(2):
            dst = ((2 * by + py) * W + (2 * bx + px)).reshape(-1, 1)
            pmats.append((dst == lane).astype(jnp.bfloat16))
    pmat = jnp.stack(pmats, axis=0)

    core = functools.partial(_core_kernel, NB=NB, Ho=Ho, Wo=Wo,
                             C_in=C_in, inner_nc=inner_nc)
    z, zsum, zssq = pl.pallas_call(
        core,
        out_shape=(jax.ShapeDtypeStruct((N, outer_nc, HW), jnp.bfloat16),
                   jax.ShapeDtypeStruct((N, outer_nc, 1), jnp.float32),
                   jax.ShapeDtypeStruct((N, outer_nc, 1), jnp.float32)),
        grid=(N // NB,),
        in_specs=[
            pl.BlockSpec((NB, C_in, HW), lambda i: (i, 0, 0)),
            pl.BlockSpec((4 * Hs * _WSP, HW), lambda i: (0, 0)),
            pl.BlockSpec((16, C_in, inner_nc), lambda i: (0, 0, 0)),
            pl.BlockSpec((16, outer_nc, inner_nc), lambda i: (0, 0, 0)),
            pl.BlockSpec((4, HoWo, HW), lambda i: (0, 0, 0)),
        ],
        out_specs=(
            pl.BlockSpec((NB, outer_nc, HW), lambda i: (i, 0, 0)),
            pl.BlockSpec((NB, outer_nc, 1), lambda i: (i, 0, 0)),
            pl.BlockSpec((NB, outer_nc, 1), lambda i: (i, 0, 0)),
        ),
        scratch_shapes=[
            pltpu.VMEM((NB, 4, Hs, _WSP, C_in), jnp.bfloat16),
            pltpu.VMEM((NB, Ho + 2, Wo + 2, inner_nc), jnp.bfloat16),
        ],
        compiler_params=pltpu.CompilerParams(
            dimension_semantics=("parallel",)),
    )(x_flat, ps2d, wd_slabs, wu_slabs, pmat)

    m = float(N * H * W)
    s = jnp.sum(zsum[:, :, 0], axis=0)
    ss = jnp.sum(zssq[:, :, 0], axis=0)
    mean = s / m
    var = jnp.maximum(ss / m - mean * mean, 0.0)
    inv_std = lax.rsqrt(var + eps)
    g = gamma.astype(jnp.float32)
    b = beta.astype(jnp.float32)
    scale = (g * inv_std).reshape(outer_nc, 1)
    shift = (b - mean * g * inv_std).reshape(outer_nc, 1)

    out_flat = pl.pallas_call(
        functools.partial(_bn_concat_kernel, C_in=C_in),
        out_shape=jax.ShapeDtypeStruct((N, C_in + outer_nc, HW), jnp.float32),
        grid=(N // NB,),
        in_specs=[
            pl.BlockSpec((NB, C_in, HW), lambda i: (i, 0, 0)),
            pl.BlockSpec((NB, outer_nc, HW), lambda i: (i, 0, 0)),
            pl.BlockSpec((outer_nc, 1), lambda i: (0, 0)),
            pl.BlockSpec((outer_nc, 1), lambda i: (0, 0)),
        ],
        out_specs=pl.BlockSpec((NB, C_in + outer_nc, HW), lambda i: (i, 0, 0)),
        compiler_params=pltpu.CompilerParams(
            dimension_semantics=("parallel",)),
    )(x_flat, z, scale, shift)

    return out_flat.reshape(N, C_in + outer_nc, H, W)
